# baseline (speedup 1.0000x reference)
import jax
import jax.numpy as jnp
from jax.experimental import pallas as pl
from jax.experimental.pallas import tpu as pltpu

B = 256
T = 200
D = 128
NSEG = 3
NUM_CLS_STATIC = 1
OUT_T = NUM_CLS_STATIC + 1 + NSEG * T + NSEG
BB = 8

_SOS, _EOS, _STP, _CLS = 1, 2, 3, 4


def _body(scale_ref, sp_ref, s0_ref, s1_ref, s2_ref, out_ref):
    tbl = sp_ref[...]
    norm = jnp.sqrt(jnp.sum(tbl * tbl, axis=1, keepdims=True))
    tbl = tbl * jnp.minimum(1.0, 1.0 / jnp.maximum(norm, 1e-12))
    cls_row = tbl[_CLS] * scale_ref[0, 0]
    out_ref[:, 0, :] = jnp.broadcast_to(cls_row, (BB, D))
    out_ref[:, 1, :] = jnp.broadcast_to(tbl[_SOS], (BB, D))
    out_ref[:, 2 : 2 + T, :] = s0_ref[...]
    out_ref[:, 2 + T, :] = jnp.broadcast_to(tbl[_STP], (BB, D))
    out_ref[:, 3 + T : 3 + 2 * T, :] = s1_ref[...]
    out_ref[:, 3 + 2 * T, :] = jnp.broadcast_to(tbl[_STP], (BB, D))
    out_ref[:, 4 + 2 * T : 4 + 3 * T, :] = s2_ref[...]
    out_ref[:, 4 + 3 * T, :] = jnp.broadcast_to(tbl[_EOS], (BB, D))


def kernel(seg0, seg1, seg2, sp_table, num_cls):
    scale = (jnp.asarray(num_cls, jnp.float32) / NUM_CLS_STATIC).reshape(1, 1)
    out = pl.pallas_call(
        _body,
        grid=(B // BB,),
        in_specs=[
            pl.BlockSpec(memory_space=pltpu.SMEM),
            pl.BlockSpec((sp_table.shape[0], D), lambda i: (0, 0)),
            pl.BlockSpec((BB, T, D), lambda i: (i, 0, 0)),
            pl.BlockSpec((BB, T, D), lambda i: (i, 0, 0)),
            pl.BlockSpec((BB, T, D), lambda i: (i, 0, 0)),
        ],
        out_specs=pl.BlockSpec((BB, OUT_T, D), lambda i: (i, 0, 0)),
        out_shape=jax.ShapeDtypeStruct((B, OUT_T, D), jnp.float32),
        compiler_params=pltpu.CompilerParams(
            dimension_semantics=("arbitrary",),
        ),
    )(scale, sp_table, seg0, seg1, seg2)
    seg_index = jnp.concatenate(
        [
            jnp.zeros(NUM_CLS_STATIC + 1 + T + 1, jnp.int32),
            jnp.ones(T + 1, jnp.int32),
            jnp.full(T + 1, 2, jnp.int32),
        ]
    )
    return out, seg_index


# speedup vs baseline: 1.0331x; 1.0331x over previous
import jax
import jax.numpy as jnp
from jax.experimental import pallas as pl
from jax.experimental.pallas import tpu as pltpu

B = 256
T = 200
D = 128
NSEG = 3
NUM_CLS_STATIC = 1
OUT_T = NUM_CLS_STATIC + 1 + NSEG * T + NSEG
BB = 32

_SOS, _EOS, _STP, _CLS = 1, 2, 3, 4


def _body(scale_ref, sp_ref, s0_ref, s1_ref, s2_ref, out_ref):
    tbl = sp_ref[...]
    norm = jnp.sqrt(jnp.sum(tbl * tbl, axis=1, keepdims=True))
    tbl = tbl * jnp.minimum(1.0, 1.0 / jnp.maximum(norm, 1e-12))
    cls_row = tbl[_CLS] * scale_ref[0, 0]
    out_ref[:, 0, :] = jnp.broadcast_to(cls_row, (BB, D))
    out_ref[:, 1, :] = jnp.broadcast_to(tbl[_SOS], (BB, D))
    out_ref[:, 2 : 2 + T, :] = s0_ref[...]
    out_ref[:, 2 + T, :] = jnp.broadcast_to(tbl[_STP], (BB, D))
    out_ref[:, 3 + T : 3 + 2 * T, :] = s1_ref[...]
    out_ref[:, 3 + 2 * T, :] = jnp.broadcast_to(tbl[_STP], (BB, D))
    out_ref[:, 4 + 2 * T : 4 + 3 * T, :] = s2_ref[...]
    out_ref[:, 4 + 3 * T, :] = jnp.broadcast_to(tbl[_EOS], (BB, D))


def kernel(seg0, seg1, seg2, sp_table, num_cls):
    scale = (jnp.asarray(num_cls, jnp.float32) / NUM_CLS_STATIC).reshape(1, 1)
    out = pl.pallas_call(
        _body,
        grid=(B // BB,),
        in_specs=[
            pl.BlockSpec(memory_space=pltpu.SMEM),
            pl.BlockSpec((sp_table.shape[0], D), lambda i: (0, 0)),
            pl.BlockSpec((BB, T, D), lambda i: (i, 0, 0)),
            pl.BlockSpec((BB, T, D), lambda i: (i, 0, 0)),
            pl.BlockSpec((BB, T, D), lambda i: (i, 0, 0)),
        ],
        out_specs=pl.BlockSpec((BB, OUT_T, D), lambda i: (i, 0, 0)),
        out_shape=jax.ShapeDtypeStruct((B, OUT_T, D), jnp.float32),
        compiler_params=pltpu.CompilerParams(
            dimension_semantics=("arbitrary",),
        ),
    )(scale, sp_table, seg0, seg1, seg2)
    seg_index = jnp.concatenate(
        [
            jnp.zeros(NUM_CLS_STATIC + 1 + T + 1, jnp.int32),
            jnp.ones(T + 1, jnp.int32),
            jnp.full(T + 1, 2, jnp.int32),
        ]
    )
    return out, seg_index
